# Initial kernel scaffold; baseline (speedup 1.0000x reference)
import jax
import jax.numpy as jnp
from jax.experimental import pallas as pl


def kernel(user_mem, item_mem, p, nei_users, nei_items, pairs_u, pairs_i):
    raise NotImplementedError("write your pallas kernel here")


# speedup vs baseline: 2.8342x; 2.8342x over previous
import functools

import jax
import jax.numpy as jnp
from jax import lax
from jax.experimental import pallas as pl
from jax.experimental.pallas import tpu as pltpu
from jax.experimental.pallas import tpu_sc as plsc

NC = 2
NS = 16
NW = NC * NS
L = 16
STEP = 1.0 / 65536.0

_SC_PARAMS = pltpu.CompilerParams(
    use_tc_tiling_on_sc=False,
    needs_layout_passes=False,
)


def _widx():
    return lax.axis_index("s") * NC + lax.axis_index("c")


@functools.partial(jax.jit, static_argnums=(2, 3))
def _build_maps(nei_users, nei_items, n_users, n_items):
    Bn = nei_users.shape[0]
    per = Bn // NW
    CH = 128
    nch = per // CH

    mesh = plsc.VectorSubcoreMesh(core_axis_name="c", subcore_axis_name="s")

    @functools.partial(
        pl.kernel,
        out_type=(jax.ShapeDtypeStruct((n_users,), jnp.int32),
                  jax.ShapeDtypeStruct((n_items,), jnp.int32)),
        mesh=mesh,
        compiler_params=_SC_PARAMS,
        scratch_types=[
            pltpu.VMEM((2 * nch, CH), jnp.int32),
            pltpu.VMEM((per,), jnp.int32),
            pltpu.SemaphoreType.DMA,
        ],
    )
    def build(nei_u_hbm, nei_i_hbm, map_u_hbm, map_i_hbm, idx2, vals, sem):
        base = _widx() * per
        for c in range(nch):
            pltpu.sync_copy(nei_u_hbm.at[pl.ds(base + c * CH, CH)], idx2.at[c])
            pltpu.sync_copy(nei_i_hbm.at[pl.ds(base + c * CH, CH)],
                            idx2.at[nch + c])
        for g in range(per // L):
            vals[pl.ds(g * L, L)] = base + g * L + lax.iota(jnp.int32, L)
        copies = []
        for c in range(nch):
            copies.append(pltpu.async_copy(
                vals.at[pl.ds(c * CH, CH)], map_u_hbm.at[idx2.at[c]], sem))
            copies.append(pltpu.async_copy(
                vals.at[pl.ds(c * CH, CH)], map_i_hbm.at[idx2.at[nch + c]],
                sem))
        for cp in copies:
            cp.wait()

    return build(nei_users, nei_items)


@jax.jit
def _score(user_mem, item_mem, p_u, p_i, map_u, map_i,
           nei_users, nei_items, pairs_u, pairs_i):
    P = pairs_u.shape[0]
    D = user_mem.shape[1]
    Bu = nei_users.shape[0]
    Bi = nei_items.shape[0]
    per = P // NW
    CH = 128
    nch = per // CH

    mesh = plsc.VectorSubcoreMesh(core_axis_name="c", subcore_axis_name="s")

    @functools.partial(
        pl.kernel,
        out_type=jax.ShapeDtypeStruct((P,), jnp.float32),
        mesh=mesh,
        compiler_params=_SC_PARAMS,
        scratch_types=[
            pltpu.VMEM((CH,), jnp.int32),
            pltpu.VMEM((CH,), jnp.int32),
            pltpu.VMEM((CH,), jnp.int32),
            pltpu.VMEM((CH,), jnp.int32),
            pltpu.VMEM((CH,), jnp.int32),
            pltpu.VMEM((CH,), jnp.int32),
            pltpu.VMEM((CH,), jnp.int32),
            pltpu.VMEM((CH,), jnp.int32),
            pltpu.VMEM((CH, 32), jnp.float32),
            pltpu.VMEM((CH, 32), jnp.float32),
            pltpu.VMEM((CH, 32), jnp.float32),
            pltpu.VMEM((CH, 32), jnp.float32),
            pltpu.VMEM((CH,), jnp.float32),
            pltpu.SemaphoreType.DMA,
        ],
    )
    def score(user_hbm, item_hbm, pu_hbm, pi_hbm, mu_hbm, mi_hbm,
              nu_hbm, ni_hbm, pru_hbm, pri_hbm, out_hbm,
              puv, piv, juv, jiv, buv, biv, nuv, niv,
              urows, irows, durows, dirows, scv, sem):
        tbase = _widx() * per

        def chunk_body(c, _):
            gb = tbase + c * CH
            pltpu.sync_copy(pru_hbm.at[pl.ds(gb, CH)], puv)
            pltpu.sync_copy(pri_hbm.at[pl.ds(gb, CH)], piv)
            cp_ju = pltpu.async_copy(mu_hbm.at[puv], juv, sem)
            cp_ji = pltpu.async_copy(mi_hbm.at[piv], jiv, sem)
            cp_ur = pltpu.async_copy(user_hbm.at[puv], urows, sem)
            cp_ir = pltpu.async_copy(item_hbm.at[piv], irows, sem)
            cp_ju.wait()
            cp_ji.wait()
            for k in range(CH // L):
                sl = pl.ds(k * L, L)
                buv[sl] = jnp.minimum(jnp.maximum(juv[sl], 0), Bu - 1)
                biv[sl] = jnp.minimum(jnp.maximum(jiv[sl], 0), Bi - 1)
            cp_nu = pltpu.async_copy(nu_hbm.at[buv], nuv, sem)
            cp_ni = pltpu.async_copy(ni_hbm.at[biv], niv, sem)
            cp_du = pltpu.async_copy(pu_hbm.at[buv], durows, sem)
            cp_di = pltpu.async_copy(pi_hbm.at[biv], dirows, sem)
            cp_nu.wait()
            cp_ni.wait()
            cp_du.wait()
            cp_di.wait()
            cp_ur.wait()
            cp_ir.wait()

            def group_body(g, _):
                sl = pl.ds(g * L, L)
                msku = jnp.where(nuv[sl] == puv[sl], STEP, 0.0)
                mski = jnp.where(niv[sl] == piv[sl], STEP, 0.0)
                acc = jnp.zeros((L,), jnp.float32)
                rvec = g * L + lax.iota(jnp.int32, L)
                for j in range(D):
                    cj = jnp.full((L,), j, jnp.int32)
                    cu = plsc.load_gather(urows, [rvec, cj])
                    du = plsc.load_gather(durows, [rvec, cj])
                    ci = plsc.load_gather(irows, [rvec, cj])
                    di = plsc.load_gather(dirows, [rvec, cj])
                    acc = acc + (cu + msku * du) * (ci + mski * di)
                scv[sl] = acc
                return 0

            lax.fori_loop(0, CH // L, group_body, 0)
            pltpu.sync_copy(scv, out_hbm.at[pl.ds(gb, CH)])
            return 0

        lax.fori_loop(0, nch, chunk_body, 0)

    return score(user_mem, item_mem, p_u, p_i, map_u, map_i,
                 nei_users, nei_items, pairs_u, pairs_i)


def kernel(user_mem, item_mem, p, nei_users, nei_items, pairs_u, pairs_i):
    d = user_mem.shape[1]
    Bu = nei_users.shape[0]
    p_u = p[: Bu * d].reshape(Bu, d)
    p_i = p[Bu * d:].reshape(-1, d)
    map_u, map_i = _build_maps(nei_users, nei_items,
                               user_mem.shape[0], item_mem.shape[0])
    return _score(user_mem, item_mem, p_u, p_i, map_u, map_i,
                  nei_users, nei_items, pairs_u, pairs_i)
